# baseline (device time: 23993 ns/iter reference)
import jax
import jax.numpy as jnp
from jax import lax
from jax.experimental import pallas as pl
from jax.experimental.pallas import tpu as pltpu

B, S, H, D = 2, 256, 8, 64
SCALE = D ** -0.5
ROWS = B * S
COLS = H * D
C = 4
W = COLS // C
BF = jnp.bfloat16

_LOCAL_ORDER = [b * H + h for p in range(C) for h in (2 * p, 2 * p + 1)
                for b in range(B)]


def kernel(Q, K, V):
    Qb = (Q * SCALE).astype(BF).reshape(ROWS, COLS)
    Kb = K.astype(BF).reshape(ROWS, COLS)
    Vb = V.astype(BF).reshape(ROWS, COLS)

    def body(
        q_ref, k_ref, v_ref, o_ref,
        krb_ref, vrb_ref, o2_ref, l_scr,
        y_send, y_recv, x_send, x_recv,
    ):
        my_x = lax.axis_index("x")
        my_y = lax.axis_index("y")
        y_peer = (my_x, 1 - my_y)
        x_peer = (1 - my_x, my_y)

        ones_s = jnp.ones((S, 1), BF)

        barrier = pltpu.get_barrier_semaphore()
        for nbr in (y_peer, x_peer):
            pl.semaphore_signal(
                barrier, inc=1, device_id=nbr,
                device_id_type=pl.DeviceIdType.MESH,
            )
        pl.semaphore_wait(barrier, 2)

        def issue_y_sends(src, dst):
            for c in range(C):
                cols_c = pl.ds(c * W, W)
                pltpu.make_async_remote_copy(
                    src_ref=src.at[:, cols_c],
                    dst_ref=dst.at[:, cols_c],
                    send_sem=y_send.at[c],
                    recv_sem=y_recv.at[c],
                    device_id=y_peer,
                    device_id_type=pl.DeviceIdType.MESH,
                ).start()

        @pl.when(my_x == 0)
        def _():
            issue_y_sends(k_ref, krb_ref)

        @pl.when(my_x == 1)
        def _():
            issue_y_sends(v_ref, vrb_ref)

        def wait_y_and_fwd(c):
            cols_c = pl.ds(c * W, W)
            pltpu.make_async_remote_copy(
                src_ref=krb_ref.at[:, cols_c],
                dst_ref=krb_ref.at[:, cols_c],
                send_sem=y_send.at[c],
                recv_sem=y_recv.at[c],
                device_id=y_peer,
                device_id_type=pl.DeviceIdType.MESH,
            ).wait_recv()

            @pl.when(my_x == 0)
            def _():
                pltpu.make_async_remote_copy(
                    src_ref=krb_ref.at[:, cols_c],
                    dst_ref=krb_ref.at[:, cols_c],
                    send_sem=x_send.at[c],
                    recv_sem=x_recv.at[c],
                    device_id=x_peer,
                    device_id_type=pl.DeviceIdType.MESH,
                ).start()

            @pl.when(my_x == 1)
            def _():
                pltpu.make_async_remote_copy(
                    src_ref=vrb_ref.at[:, cols_c],
                    dst_ref=vrb_ref.at[:, cols_c],
                    send_sem=x_send.at[c],
                    recv_sem=x_recv.at[c],
                    device_id=x_peer,
                    device_id_type=pl.DeviceIdType.MESH,
                ).start()

        def wait_x(c):
            cols_c = pl.ds(c * W, W)
            pltpu.make_async_remote_copy(
                src_ref=vrb_ref.at[:, cols_c],
                dst_ref=vrb_ref.at[:, cols_c],
                send_sem=x_send.at[c],
                recv_sem=x_recv.at[c],
                device_id=x_peer,
                device_id_type=pl.DeviceIdType.MESH,
            ).wait_recv()

        def local_unit(g):
            b, h = divmod(g, H)
            rows = pl.ds(b * S, S)
            cols = pl.ds(h * D, D)
            q = q_ref[rows, cols]
            s = lax.dot_general(
                q, k_ref[rows, cols], (((1,), (1,)), ((), ())),
                preferred_element_type=jnp.float32,
            )
            p = jnp.exp(s.astype(BF))
            l_scr[g] = lax.dot_general(
                p, ones_s, (((1,), (0,)), ((), ())),
                preferred_element_type=jnp.float32,
            )
            o2_ref[rows, cols] = lax.dot_general(
                p, v_ref[rows, cols], (((1,), (0,)), ((), ())),
                preferred_element_type=jnp.float32,
            )

        def remote_unit(b, h):
            g = b * H + h
            rows = pl.ds(b * S, S)
            cols = pl.ds(h * D, D)
            q = q_ref[rows, cols]
            s = lax.dot_general(
                q, krb_ref[rows, cols], (((1,), (1,)), ((), ())),
                preferred_element_type=jnp.float32,
            )
            p = jnp.exp(s.astype(BF))
            l = l_scr[g] + lax.dot_general(
                p, ones_s, (((1,), (0,)), ((), ())),
                preferred_element_type=jnp.float32,
            )
            o = o2_ref[rows, cols] + lax.dot_general(
                p, vrb_ref[rows, cols], (((1,), (0,)), ((), ())),
                preferred_element_type=jnp.float32,
            )
            o_ref[rows, cols] = (o / l).astype(BF)

        for i in range(6):
            local_unit(_LOCAL_ORDER[i])
        for c in range(C):
            wait_y_and_fwd(c)
            local_unit(_LOCAL_ORDER[6 + 2 * c])
            local_unit(_LOCAL_ORDER[7 + 2 * c])
        for i in range(14, B * H):
            local_unit(_LOCAL_ORDER[i])
        for c in range(C):
            wait_x(c)
            for h in (2 * c, 2 * c + 1):
                for b in range(B):
                    remote_unit(b, h)

        for c in range(C):
            cols_c = pl.ds(c * W, W)
            for s_sem, peer in ((y_send, y_peer), (x_send, x_peer)):
                pltpu.make_async_remote_copy(
                    src_ref=krb_ref.at[:, cols_c],
                    dst_ref=krb_ref.at[:, cols_c],
                    send_sem=s_sem.at[c],
                    recv_sem=y_recv.at[c],
                    device_id=peer,
                    device_id_type=pl.DeviceIdType.MESH,
                ).wait_send()

    out = pl.pallas_call(
        body,
        out_shape=jax.ShapeDtypeStruct((ROWS, COLS), BF),
        in_specs=[pl.BlockSpec(memory_space=pltpu.VMEM)] * 3,
        out_specs=pl.BlockSpec(memory_space=pltpu.VMEM),
        scratch_shapes=[
            pltpu.VMEM((ROWS, COLS), BF),
            pltpu.VMEM((ROWS, COLS), BF),
            pltpu.VMEM((ROWS, COLS), jnp.float32),
            pltpu.VMEM((B * H, S, 1), jnp.float32),
            pltpu.SemaphoreType.DMA((C,)),
            pltpu.SemaphoreType.DMA((C,)),
            pltpu.SemaphoreType.DMA((C,)),
            pltpu.SemaphoreType.DMA((C,)),
        ],
        compiler_params=pltpu.CompilerParams(collective_id=0),
    )(Qb, Kb, Vb)
    return out.reshape(B, S, H, D).astype(jnp.float32)


# device time: 23325 ns/iter; 1.0286x vs baseline; 1.0286x over previous
import jax
import jax.numpy as jnp
from jax import lax
from jax.experimental import pallas as pl
from jax.experimental.pallas import tpu as pltpu

B, S, H, D = 2, 256, 8, 64
SCALE = D ** -0.5
ROWS = B * S
COLS = H * D
C = 4
W = COLS // C
BF = jnp.bfloat16

_LOCAL_ORDER = [b * H + h for p in range(C) for h in (2 * p, 2 * p + 1)
                for b in range(B)]


def kernel(Q, K, V):
    Qb = (Q * SCALE).astype(BF).reshape(ROWS, COLS)
    Kb = K.astype(BF).reshape(ROWS, COLS)
    Vb = V.astype(BF).reshape(ROWS, COLS)

    def body(
        q_ref, k_ref, v_ref, o_ref,
        krb_ref, vrb_ref, o2_ref, l_scr,
        y_send, y_recv, x_send, x_recv,
    ):
        my_x = lax.axis_index("x")
        my_y = lax.axis_index("y")
        y_peer = (my_x, 1 - my_y)
        x_peer = (1 - my_x, my_y)

        ones_s = jnp.ones((S, 1), BF)

        barrier = pltpu.get_barrier_semaphore()
        for nbr in (y_peer, x_peer):
            pl.semaphore_signal(
                barrier, inc=1, device_id=nbr,
                device_id_type=pl.DeviceIdType.MESH,
            )
        pl.semaphore_wait(barrier, 2)

        def issue_y_sends(src, dst):
            for c in range(C):
                cols_c = pl.ds(c * W, W)
                pltpu.make_async_remote_copy(
                    src_ref=src.at[:, cols_c],
                    dst_ref=dst.at[:, cols_c],
                    send_sem=y_send.at[c],
                    recv_sem=y_recv.at[c],
                    device_id=y_peer,
                    device_id_type=pl.DeviceIdType.MESH,
                ).start()

        @pl.when(my_x == 0)
        def _():
            issue_y_sends(k_ref, krb_ref)

        @pl.when(my_x == 1)
        def _():
            issue_y_sends(v_ref, vrb_ref)

        def wait_y_and_fwd(c):
            cols_c = pl.ds(c * W, W)
            pltpu.make_async_remote_copy(
                src_ref=krb_ref.at[:, cols_c],
                dst_ref=krb_ref.at[:, cols_c],
                send_sem=y_send.at[c],
                recv_sem=y_recv.at[c],
                device_id=y_peer,
                device_id_type=pl.DeviceIdType.MESH,
            ).wait_recv()

            @pl.when(my_x == 0)
            def _():
                pltpu.make_async_remote_copy(
                    src_ref=krb_ref.at[:, cols_c],
                    dst_ref=krb_ref.at[:, cols_c],
                    send_sem=x_send.at[c],
                    recv_sem=x_recv.at[c],
                    device_id=x_peer,
                    device_id_type=pl.DeviceIdType.MESH,
                ).start()

            @pl.when(my_x == 1)
            def _():
                pltpu.make_async_remote_copy(
                    src_ref=vrb_ref.at[:, cols_c],
                    dst_ref=vrb_ref.at[:, cols_c],
                    send_sem=x_send.at[c],
                    recv_sem=x_recv.at[c],
                    device_id=x_peer,
                    device_id_type=pl.DeviceIdType.MESH,
                ).start()

        def wait_x(c):
            cols_c = pl.ds(c * W, W)
            pltpu.make_async_remote_copy(
                src_ref=vrb_ref.at[:, cols_c],
                dst_ref=vrb_ref.at[:, cols_c],
                send_sem=x_send.at[c],
                recv_sem=x_recv.at[c],
                device_id=x_peer,
                device_id_type=pl.DeviceIdType.MESH,
            ).wait_recv()

        def local_unit(g):
            b, h = divmod(g, H)
            rows = pl.ds(b * S, S)
            cols = pl.ds(h * D, D)
            q = q_ref[rows, cols]
            s = lax.dot_general(
                q, k_ref[rows, cols], (((1,), (1,)), ((), ())),
                preferred_element_type=jnp.float32,
            )
            p = jnp.exp(s.astype(BF))
            l_scr[g] = lax.dot_general(
                p, ones_s, (((1,), (0,)), ((), ())),
                preferred_element_type=jnp.float32,
            )
            o2_ref[rows, cols] = lax.dot_general(
                p, v_ref[rows, cols], (((1,), (0,)), ((), ())),
                preferred_element_type=jnp.float32,
            )

        def remote_unit(b, h):
            g = b * H + h
            rows = pl.ds(b * S, S)
            cols = pl.ds(h * D, D)
            q = q_ref[rows, cols]
            s = lax.dot_general(
                q, krb_ref[rows, cols], (((1,), (1,)), ((), ())),
                preferred_element_type=jnp.float32,
            )
            p = jnp.exp(s.astype(BF))
            l = l_scr[g] + lax.dot_general(
                p, ones_s, (((1,), (0,)), ((), ())),
                preferred_element_type=jnp.float32,
            )
            o = o2_ref[rows, cols] + lax.dot_general(
                p, vrb_ref[rows, cols], (((1,), (0,)), ((), ())),
                preferred_element_type=jnp.float32,
            )
            o_ref[rows, cols] = (o / l).astype(BF)

        def remote_chunk(c):
            for h in (2 * c, 2 * c + 1):
                for b in range(B):
                    remote_unit(b, h)

        for i in range(6):
            local_unit(_LOCAL_ORDER[i])
        for c in range(3):
            wait_y_and_fwd(c)
            local_unit(_LOCAL_ORDER[6 + 2 * c])
            local_unit(_LOCAL_ORDER[7 + 2 * c])
        wait_x(0)
        remote_chunk(0)
        wait_y_and_fwd(3)
        for i in range(12, B * H):
            local_unit(_LOCAL_ORDER[i])
        for c in range(1, C):
            wait_x(c)
            remote_chunk(c)

        for c in range(C):
            cols_c = pl.ds(c * W, W)
            for s_sem, peer in ((y_send, y_peer), (x_send, x_peer)):
                pltpu.make_async_remote_copy(
                    src_ref=krb_ref.at[:, cols_c],
                    dst_ref=krb_ref.at[:, cols_c],
                    send_sem=s_sem.at[c],
                    recv_sem=y_recv.at[c],
                    device_id=peer,
                    device_id_type=pl.DeviceIdType.MESH,
                ).wait_send()

    out = pl.pallas_call(
        body,
        out_shape=jax.ShapeDtypeStruct((ROWS, COLS), BF),
        in_specs=[pl.BlockSpec(memory_space=pltpu.VMEM)] * 3,
        out_specs=pl.BlockSpec(memory_space=pltpu.VMEM),
        scratch_shapes=[
            pltpu.VMEM((ROWS, COLS), BF),
            pltpu.VMEM((ROWS, COLS), BF),
            pltpu.VMEM((ROWS, COLS), jnp.float32),
            pltpu.VMEM((B * H, S, 1), jnp.float32),
            pltpu.SemaphoreType.DMA((C,)),
            pltpu.SemaphoreType.DMA((C,)),
            pltpu.SemaphoreType.DMA((C,)),
            pltpu.SemaphoreType.DMA((C,)),
        ],
        compiler_params=pltpu.CompilerParams(collective_id=0),
    )(Qb, Kb, Vb)
    return out.reshape(B, S, H, D).astype(jnp.float32)
